# Initial kernel scaffold; baseline (speedup 1.0000x reference)
import functools

import jax
import jax.numpy as jnp
from jax import lax
from jax.experimental import pallas as pl
from jax.experimental.pallas import tpu as pltpu
from jax.experimental.pallas import tpu_sc as plsc

N = 10000
E = 320000
D = 128
G = 128

NC = 2
NS = 16
NW = NC * NS
EP = E // NW
CHUNK = 80
ITERS = EP // CHUNK
RPT = N // NS

_HIGH = lax.Precision.HIGHEST


def _sc_aggregate(h, src, dst, zeros):
    mesh = plsc.VectorSubcoreMesh(core_axis_name="c", subcore_axis_name="s",
                                  num_cores=NC, num_subcores=NS)

    @functools.partial(
        pl.kernel,
        out_type=jax.ShapeDtypeStruct((NC, N, D), jnp.float32),
        mesh=mesh,
        scratch_types=[
            pltpu.VMEM((CHUNK,), jnp.int32),
            pltpu.VMEM((CHUNK,), jnp.int32),
            pltpu.VMEM((CHUNK, D), jnp.float32),
            pltpu.VMEM_SHARED((N, D), jnp.float32),
            pltpu.SemaphoreType.DMA,
        ],
    )
    def k(h_hbm, src_hbm, dst_hbm, z_hbm, out_hbm, src_v, dst_v, rows_v,
          agg_sh, sem):
        c = lax.axis_index("c")
        s = lax.axis_index("s")
        wid = s * NC + c

        r0 = s * RPT

        @pl.when(c == 0)
        def _():
            pltpu.sync_copy(h_hbm.at[pl.ds(r0, RPT)], agg_sh.at[pl.ds(r0, RPT)])

        @pl.when(c != 0)
        def _():
            pltpu.sync_copy(z_hbm.at[pl.ds(r0, RPT)], agg_sh.at[pl.ds(r0, RPT)])

        plsc.subcore_barrier()

        base = wid * EP

        def body(i, carry):
            off = base + i * CHUNK
            pltpu.sync_copy(src_hbm.at[pl.ds(off, CHUNK)], src_v)
            pltpu.sync_copy(dst_hbm.at[pl.ds(off, CHUNK)], dst_v)
            pltpu.async_copy(h_hbm.at[src_v], rows_v, sem).wait()
            pltpu.sync_copy(rows_v, agg_sh.at[dst_v], add=True)
            return carry

        lax.fori_loop(0, ITERS, body, 0)

        plsc.subcore_barrier()
        pltpu.sync_copy(agg_sh.at[pl.ds(r0, RPT)], out_hbm.at[c, pl.ds(r0, RPT)])

    return k(h, src, dst, zeros)


def _tc_mlp(agg, W1, b1, W2, b2):
    BLK = 2000
    grid = (N // BLK,)

    def body(a_ref, w1_ref, b1_ref, w2_ref, b2_ref, o_ref):
        y = a_ref[0] + a_ref[1]
        z = jnp.maximum(
            jnp.dot(y, w1_ref[...], preferred_element_type=jnp.float32,
                    precision=_HIGH) + b1_ref[...], 0.0)
        o_ref[...] = jnp.dot(z, w2_ref[...], preferred_element_type=jnp.float32,
                             precision=_HIGH) + b2_ref[...]

    return pl.pallas_call(
        body,
        grid=grid,
        in_specs=[
            pl.BlockSpec((NC, BLK, D), lambda i: (0, i, 0)),
            pl.BlockSpec((D, D), lambda i: (0, 0)),
            pl.BlockSpec((1, D), lambda i: (0, 0)),
            pl.BlockSpec((D, D), lambda i: (0, 0)),
            pl.BlockSpec((1, D), lambda i: (0, 0)),
        ],
        out_specs=pl.BlockSpec((BLK, D), lambda i: (i, 0)),
        out_shape=jax.ShapeDtypeStruct((N, D), jnp.float32),
    )(agg, W1, b1.reshape(1, D), W2, b2.reshape(1, D))


def _tc_pool_mlp(h, batch3, mW1, mb1, mW2, mb2):
    BLK = 2000
    nblk = N // BLK

    def body(h_ref, bat_ref, w1_ref, b1_ref, w2_ref, b2_ref, o_ref,
             sums, counts):
        i = pl.program_id(0)

        @pl.when(i == 0)
        def _():
            sums[...] = jnp.zeros_like(sums)
            counts[...] = jnp.zeros_like(counts)

        ids = bat_ref[0, 0, :]
        onehot = (ids[None, :] == lax.broadcasted_iota(
            jnp.int32, (G, BLK), 0)).astype(jnp.float32)
        sums[...] += jnp.dot(onehot, h_ref[...],
                             preferred_element_type=jnp.float32,
                             precision=_HIGH)
        counts[...] += jnp.sum(onehot, axis=1, keepdims=True)

        @pl.when(i == nblk - 1)
        def _():
            pooled = sums[...] / jnp.maximum(counts[...], 1.0)
            z = jnp.maximum(
                jnp.dot(pooled, w1_ref[...], preferred_element_type=jnp.float32,
                        precision=_HIGH) + b1_ref[...], 0.0)
            o_ref[...] = jnp.dot(z, w2_ref[...],
                                 preferred_element_type=jnp.float32,
                                 precision=_HIGH) + b2_ref[...]

    return pl.pallas_call(
        body,
        grid=(nblk,),
        in_specs=[
            pl.BlockSpec((BLK, D), lambda i: (i, 0)),
            pl.BlockSpec((1, 1, BLK), lambda i: (i, 0, 0)),
            pl.BlockSpec((D, D), lambda i: (0, 0)),
            pl.BlockSpec((1, D), lambda i: (0, 0)),
            pl.BlockSpec((D, G), lambda i: (0, 0)),
            pl.BlockSpec((1, G), lambda i: (0, 0)),
        ],
        out_specs=pl.BlockSpec((G, G), lambda i: (0, 0)),
        out_shape=jax.ShapeDtypeStruct((G, G), jnp.float32),
        scratch_shapes=[
            pltpu.VMEM((G, D), jnp.float32),
            pltpu.VMEM((G, 1), jnp.float32),
        ],
    )(h, batch3, mW1, mb1.reshape(1, D), mW2, mb2.reshape(1, G))


def kernel(x, edge_index, batch, conv0_W1, conv0_b1, conv0_W2, conv0_b2,
           conv1_W1, conv1_b1, conv1_W2, conv1_b2,
           conv2_W1, conv2_b1, conv2_W2, conv2_b2,
           mlp_W1, mlp_b1, mlp_W2, mlp_b2):
    src = edge_index[0]
    dst = edge_index[1]
    zeros = jnp.zeros((N, D), dtype=jnp.float32)
    batch3 = batch.reshape(N // 2000, 1, 2000)

    h = x
    for (W1, b1, W2, b2) in (
        (conv0_W1, conv0_b1, conv0_W2, conv0_b2),
        (conv1_W1, conv1_b1, conv1_W2, conv1_b2),
        (conv2_W1, conv2_b1, conv2_W2, conv2_b2),
    ):
        agg = _sc_aggregate(h, src, dst, zeros)
        h = _tc_mlp(agg, W1, b1, W2, b2)

    return _tc_pool_mlp(h, batch3, mlp_W1, mlp_b1, mlp_W2, mlp_b2)


# speedup vs baseline: 4.9967x; 4.9967x over previous
import functools

import jax
import jax.numpy as jnp
from jax import lax
from jax.experimental import pallas as pl
from jax.experimental.pallas import tpu as pltpu
from jax.experimental.pallas import tpu_sc as plsc

N = 10000
E = 320000
D = 128
G = 128

NC = 2
NS = 16
NW = NC * NS
EP = E // NW
CHUNK = 80
ITERS = EP // CHUNK
ROWC = 400
NRC = N // ROWC

_HIGH = lax.Precision.HIGHEST


def _sc_aggregate(h, src, dst, zeros):
    mesh = plsc.VectorSubcoreMesh(core_axis_name="c", subcore_axis_name="s",
                                  num_cores=NC, num_subcores=NS)

    @functools.partial(
        pl.kernel,
        out_type=jax.ShapeDtypeStruct((NC, N, D), jnp.float32),
        mesh=mesh,
        scratch_types=[
            pltpu.VMEM((CHUNK,), jnp.int32),
            pltpu.VMEM((CHUNK,), jnp.int32),
            pltpu.VMEM((CHUNK, D), jnp.float32),
            pltpu.VMEM_SHARED((N, D), jnp.float32),
            pltpu.SemaphoreType.DMA,
        ],
    )
    def k(h_hbm, src_hbm, dst_hbm, z_hbm, out_hbm, src_v, dst_v, rows_v,
          agg_sh, sem):
        c = lax.axis_index("c")
        s = lax.axis_index("s")
        wid = s * NC + c

        for kk in range(2):
            cid = s + NS * kk

            @pl.when(cid < NRC)
            def _():
                off = cid * ROWC

                @pl.when(c == 0)
                def _():
                    pltpu.sync_copy(h_hbm.at[pl.ds(off, ROWC)],
                                    agg_sh.at[pl.ds(off, ROWC)])

                @pl.when(c != 0)
                def _():
                    pltpu.sync_copy(z_hbm.at[pl.ds(off, ROWC)],
                                    agg_sh.at[pl.ds(off, ROWC)])

        plsc.subcore_barrier()

        base = wid * EP

        def body(i, carry):
            off = base + i * CHUNK
            pltpu.sync_copy(src_hbm.at[pl.ds(off, CHUNK)], src_v)
            pltpu.sync_copy(dst_hbm.at[pl.ds(off, CHUNK)], dst_v)
            pltpu.async_copy(h_hbm.at[src_v], rows_v, sem).wait()
            pltpu.sync_copy(rows_v, agg_sh.at[dst_v], add=True)
            return carry

        lax.fori_loop(0, ITERS, body, 0)

        plsc.subcore_barrier()

        for kk in range(2):
            cid = s + NS * kk

            @pl.when(cid < NRC)
            def _():
                off = cid * ROWC
                pltpu.sync_copy(agg_sh.at[pl.ds(off, ROWC)],
                                out_hbm.at[c, pl.ds(off, ROWC)])

    return k(h, src, dst, zeros)


def _tc_mlp(agg, W1, b1, W2, b2):
    BLK = 2000
    grid = (N // BLK,)

    def body(a_ref, w1_ref, b1_ref, w2_ref, b2_ref, o_ref):
        y = a_ref[0] + a_ref[1]
        z = jnp.maximum(
            jnp.dot(y, w1_ref[...], preferred_element_type=jnp.float32,
                    precision=_HIGH) + b1_ref[...], 0.0)
        o_ref[...] = jnp.dot(z, w2_ref[...], preferred_element_type=jnp.float32,
                             precision=_HIGH) + b2_ref[...]

    return pl.pallas_call(
        body,
        grid=grid,
        in_specs=[
            pl.BlockSpec((NC, BLK, D), lambda i: (0, i, 0)),
            pl.BlockSpec((D, D), lambda i: (0, 0)),
            pl.BlockSpec((1, D), lambda i: (0, 0)),
            pl.BlockSpec((D, D), lambda i: (0, 0)),
            pl.BlockSpec((1, D), lambda i: (0, 0)),
        ],
        out_specs=pl.BlockSpec((BLK, D), lambda i: (i, 0)),
        out_shape=jax.ShapeDtypeStruct((N, D), jnp.float32),
    )(agg, W1, b1.reshape(1, D), W2, b2.reshape(1, D))


def _tc_pool_mlp(h, batch3, mW1, mb1, mW2, mb2):
    BLK = 2000
    nblk = N // BLK

    def body(h_ref, bat_ref, w1_ref, b1_ref, w2_ref, b2_ref, o_ref,
             sums, counts):
        i = pl.program_id(0)

        @pl.when(i == 0)
        def _():
            sums[...] = jnp.zeros_like(sums)
            counts[...] = jnp.zeros_like(counts)

        ids = bat_ref[0, 0, :]
        onehot = (ids[None, :] == lax.broadcasted_iota(
            jnp.int32, (G, BLK), 0)).astype(jnp.float32)
        sums[...] += jnp.dot(onehot, h_ref[...],
                             preferred_element_type=jnp.float32,
                             precision=_HIGH)
        counts[...] += jnp.sum(onehot, axis=1, keepdims=True)

        @pl.when(i == nblk - 1)
        def _():
            pooled = sums[...] / jnp.maximum(counts[...], 1.0)
            z = jnp.maximum(
                jnp.dot(pooled, w1_ref[...], preferred_element_type=jnp.float32,
                        precision=_HIGH) + b1_ref[...], 0.0)
            o_ref[...] = jnp.dot(z, w2_ref[...],
                                 preferred_element_type=jnp.float32,
                                 precision=_HIGH) + b2_ref[...]

    return pl.pallas_call(
        body,
        grid=(nblk,),
        in_specs=[
            pl.BlockSpec((BLK, D), lambda i: (i, 0)),
            pl.BlockSpec((1, 1, BLK), lambda i: (i, 0, 0)),
            pl.BlockSpec((D, D), lambda i: (0, 0)),
            pl.BlockSpec((1, D), lambda i: (0, 0)),
            pl.BlockSpec((D, G), lambda i: (0, 0)),
            pl.BlockSpec((1, G), lambda i: (0, 0)),
        ],
        out_specs=pl.BlockSpec((G, G), lambda i: (0, 0)),
        out_shape=jax.ShapeDtypeStruct((G, G), jnp.float32),
        scratch_shapes=[
            pltpu.VMEM((G, D), jnp.float32),
            pltpu.VMEM((G, 1), jnp.float32),
        ],
    )(h, batch3, mW1, mb1.reshape(1, D), mW2, mb2.reshape(1, G))


def kernel(x, edge_index, batch, conv0_W1, conv0_b1, conv0_W2, conv0_b2,
           conv1_W1, conv1_b1, conv1_W2, conv1_b2,
           conv2_W1, conv2_b1, conv2_W2, conv2_b2,
           mlp_W1, mlp_b1, mlp_W2, mlp_b2):
    src = edge_index[0]
    dst = edge_index[1]
    zeros = jnp.zeros((N, D), dtype=jnp.float32)
    batch3 = batch.reshape(N // 2000, 1, 2000)

    h = x
    for (W1, b1, W2, b2) in (
        (conv0_W1, conv0_b1, conv0_W2, conv0_b2),
        (conv1_W1, conv1_b1, conv1_W2, conv1_b2),
        (conv2_W1, conv2_b1, conv2_W2, conv2_b2),
    ):
        agg = _sc_aggregate(h, src, dst, zeros)
        h = _tc_mlp(agg, W1, b1, W2, b2)

    return _tc_pool_mlp(h, batch3, mlp_W1, mlp_b1, mlp_W2, mlp_b2)


# speedup vs baseline: 11.1521x; 2.2319x over previous
import functools

import jax
import jax.numpy as jnp
from jax import lax
from jax.experimental import pallas as pl
from jax.experimental.pallas import tpu as pltpu
from jax.experimental.pallas import tpu_sc as plsc

N = 10000
E = 320000
D = 128
G = 128

NC = 2
NS = 16
NW = NC * NS
EP = E // NW
CHUNK = 80
ITERS = EP // CHUNK
ROWC = 400
NRC = N // ROWC

_HIGH = lax.Precision.HIGHEST


def _sc_aggregate(h, src, dst3, zeros):
    mesh = plsc.VectorSubcoreMesh(core_axis_name="c", subcore_axis_name="s",
                                  num_cores=NC, num_subcores=NS)

    @functools.partial(
        pl.kernel,
        out_type=jax.ShapeDtypeStruct((NC, N, D), jnp.float32),
        mesh=mesh,
        scratch_types=[
            pltpu.VMEM((EP,), jnp.int32),
            pltpu.VMEM((ITERS, CHUNK), jnp.int32),
            pltpu.VMEM((CHUNK, D), jnp.float32),
            pltpu.VMEM((CHUNK, D), jnp.float32),
            pltpu.VMEM_SHARED((N, D), jnp.float32),
            pltpu.SemaphoreType.DMA,
            pltpu.SemaphoreType.DMA,
        ],
    )
    def k(h_hbm, src_hbm, dst_hbm, z_hbm, out_hbm, src_v, dst_v, rows_a,
          rows_b, agg_sh, sem_a, sem_b):
        c = lax.axis_index("c")
        s = lax.axis_index("s")
        wid = s * NC + c
        base = wid * EP

        pltpu.sync_copy(src_hbm.at[pl.ds(base, EP)], src_v)
        pltpu.sync_copy(dst_hbm.at[wid], dst_v)

        for kk in range(2):
            cid = s + NS * kk

            @pl.when(cid < NRC)
            def _():
                off = cid * ROWC

                @pl.when(c == 0)
                def _():
                    pltpu.sync_copy(h_hbm.at[pl.ds(off, ROWC)],
                                    agg_sh.at[pl.ds(off, ROWC)])

                @pl.when(c != 0)
                def _():
                    pltpu.sync_copy(z_hbm.at[pl.ds(off, ROWC)],
                                    agg_sh.at[pl.ds(off, ROWC)])

        plsc.subcore_barrier()

        def issue(i, buf, sem):
            pltpu.async_copy(h_hbm.at[src_v.at[pl.ds(i * CHUNK, CHUNK)]],
                             buf, sem)

        def wait(i, buf, sem):
            pltpu.make_async_copy(h_hbm.at[src_v.at[pl.ds(i * CHUNK, CHUNK)]],
                                  buf, sem).wait()

        def scatter(i, buf):
            pltpu.sync_copy(buf, agg_sh.at[dst_v.at[i]], add=True)

        issue(0, rows_a, sem_a)

        def body(j, carry):
            i0 = 2 * j
            issue(i0 + 1, rows_b, sem_b)
            wait(i0, rows_a, sem_a)
            scatter(i0, rows_a)
            issue(i0 + 2, rows_a, sem_a)
            wait(i0 + 1, rows_b, sem_b)
            scatter(i0 + 1, rows_b)
            return carry

        lax.fori_loop(0, (ITERS - 1) // 2, body, 0)
        wait(ITERS - 1, rows_a, sem_a)
        scatter(ITERS - 1, rows_a)

        plsc.subcore_barrier()

        for kk in range(2):
            cid = s + NS * kk

            @pl.when(cid < NRC)
            def _():
                off = cid * ROWC
                pltpu.sync_copy(agg_sh.at[pl.ds(off, ROWC)],
                                out_hbm.at[c, pl.ds(off, ROWC)])

    return k(h, src, dst3, zeros)


def _tc_mlp(agg, W1, b1, W2, b2):
    BLK = 2000
    grid = (N // BLK,)

    def body(a_ref, w1_ref, b1_ref, w2_ref, b2_ref, o_ref):
        y = a_ref[0] + a_ref[1]
        z = jnp.maximum(
            jnp.dot(y, w1_ref[...], preferred_element_type=jnp.float32,
                    precision=_HIGH) + b1_ref[...], 0.0)
        o_ref[...] = jnp.dot(z, w2_ref[...], preferred_element_type=jnp.float32,
                             precision=_HIGH) + b2_ref[...]

    return pl.pallas_call(
        body,
        grid=grid,
        in_specs=[
            pl.BlockSpec((NC, BLK, D), lambda i: (0, i, 0)),
            pl.BlockSpec((D, D), lambda i: (0, 0)),
            pl.BlockSpec((1, D), lambda i: (0, 0)),
            pl.BlockSpec((D, D), lambda i: (0, 0)),
            pl.BlockSpec((1, D), lambda i: (0, 0)),
        ],
        out_specs=pl.BlockSpec((BLK, D), lambda i: (i, 0)),
        out_shape=jax.ShapeDtypeStruct((N, D), jnp.float32),
    )(agg, W1, b1.reshape(1, D), W2, b2.reshape(1, D))


def _tc_pool_mlp(h, batch3, mW1, mb1, mW2, mb2):
    BLK = 2000
    nblk = N // BLK

    def body(h_ref, bat_ref, w1_ref, b1_ref, w2_ref, b2_ref, o_ref,
             sums, counts):
        i = pl.program_id(0)

        @pl.when(i == 0)
        def _():
            sums[...] = jnp.zeros_like(sums)
            counts[...] = jnp.zeros_like(counts)

        ids = bat_ref[0, 0, :]
        onehot = (ids[None, :] == lax.broadcasted_iota(
            jnp.int32, (G, BLK), 0)).astype(jnp.float32)
        sums[...] += jnp.dot(onehot, h_ref[...],
                             preferred_element_type=jnp.float32,
                             precision=_HIGH)
        counts[...] += jnp.sum(onehot, axis=1, keepdims=True)

        @pl.when(i == nblk - 1)
        def _():
            pooled = sums[...] / jnp.maximum(counts[...], 1.0)
            z = jnp.maximum(
                jnp.dot(pooled, w1_ref[...], preferred_element_type=jnp.float32,
                        precision=_HIGH) + b1_ref[...], 0.0)
            o_ref[...] = jnp.dot(z, w2_ref[...],
                                 preferred_element_type=jnp.float32,
                                 precision=_HIGH) + b2_ref[...]

    return pl.pallas_call(
        body,
        grid=(nblk,),
        in_specs=[
            pl.BlockSpec((BLK, D), lambda i: (i, 0)),
            pl.BlockSpec((1, 1, BLK), lambda i: (i, 0, 0)),
            pl.BlockSpec((D, D), lambda i: (0, 0)),
            pl.BlockSpec((1, D), lambda i: (0, 0)),
            pl.BlockSpec((D, G), lambda i: (0, 0)),
            pl.BlockSpec((1, G), lambda i: (0, 0)),
        ],
        out_specs=pl.BlockSpec((G, G), lambda i: (0, 0)),
        out_shape=jax.ShapeDtypeStruct((G, G), jnp.float32),
        scratch_shapes=[
            pltpu.VMEM((G, D), jnp.float32),
            pltpu.VMEM((G, 1), jnp.float32),
        ],
    )(h, batch3, mW1, mb1.reshape(1, D), mW2, mb2.reshape(1, G))


def kernel(x, edge_index, batch, conv0_W1, conv0_b1, conv0_W2, conv0_b2,
           conv1_W1, conv1_b1, conv1_W2, conv1_b2,
           conv2_W1, conv2_b1, conv2_W2, conv2_b2,
           mlp_W1, mlp_b1, mlp_W2, mlp_b2):
    src = edge_index[0]
    dst3 = edge_index[1].reshape(NW, ITERS, CHUNK)
    zeros = jnp.zeros((N, D), dtype=jnp.float32)
    batch3 = batch.reshape(N // 2000, 1, 2000)

    h = x
    for (W1, b1, W2, b2) in (
        (conv0_W1, conv0_b1, conv0_W2, conv0_b2),
        (conv1_W1, conv1_b1, conv1_W2, conv1_b2),
        (conv2_W1, conv2_b1, conv2_W2, conv2_b2),
    ):
        agg = _sc_aggregate(h, src, dst3, zeros)
        h = _tc_mlp(agg, W1, b1, W2, b2)

    return _tc_pool_mlp(h, batch3, mlp_W1, mlp_b1, mlp_W2, mlp_b2)


# speedup vs baseline: 11.2313x; 1.0071x over previous
import functools

import jax
import jax.numpy as jnp
from jax import lax
from jax.experimental import pallas as pl
from jax.experimental.pallas import tpu as pltpu
from jax.experimental.pallas import tpu_sc as plsc

N = 10000
E = 320000
D = 128
G = 128

NC = 2
NS = 16
NW = NC * NS
EP = E // NW
CHUNK = 80
ITERS = EP // CHUNK
ROWC = 400
NRC = N // ROWC

_HIGH = lax.Precision.HIGHEST


def _sc_aggregate(h, src, dst3, zeros):
    mesh = plsc.VectorSubcoreMesh(core_axis_name="c", subcore_axis_name="s",
                                  num_cores=NC, num_subcores=NS)

    @functools.partial(
        pl.kernel,
        out_type=jax.ShapeDtypeStruct((NC, N, D), jnp.float32),
        mesh=mesh,
        scratch_types=[
            pltpu.VMEM((EP,), jnp.int32),
            pltpu.VMEM((ITERS, CHUNK), jnp.int32),
            pltpu.VMEM((CHUNK, D), jnp.float32),
            pltpu.VMEM((CHUNK, D), jnp.float32),
            pltpu.VMEM_SHARED((N, D), jnp.float32),
            pltpu.SemaphoreType.DMA,
            pltpu.SemaphoreType.DMA,
        ],
    )
    def k(h_hbm, src_hbm, dst_hbm, z_hbm, out_hbm, src_v, dst_v, rows_a,
          rows_b, agg_sh, sem_a, sem_b):
        c = lax.axis_index("c")
        s = lax.axis_index("s")
        wid = s * NC + c
        base = wid * EP

        pltpu.sync_copy(src_hbm.at[pl.ds(base, EP)], src_v)
        pltpu.sync_copy(dst_hbm.at[wid], dst_v)

        for kk in range(2):
            cid = s + NS * kk

            @pl.when(cid < NRC)
            def _():
                off = cid * ROWC

                @pl.when(c == 0)
                def _():
                    pltpu.sync_copy(h_hbm.at[pl.ds(off, ROWC)],
                                    agg_sh.at[pl.ds(off, ROWC)])

                @pl.when(c != 0)
                def _():
                    pltpu.sync_copy(z_hbm.at[pl.ds(off, ROWC)],
                                    agg_sh.at[pl.ds(off, ROWC)])

        plsc.subcore_barrier()

        def issue(i, buf, sem):
            pltpu.async_copy(h_hbm.at[src_v.at[pl.ds(i * CHUNK, CHUNK)]],
                             buf, sem)

        def wait(i, buf, sem):
            pltpu.make_async_copy(h_hbm.at[src_v.at[pl.ds(i * CHUNK, CHUNK)]],
                                  buf, sem).wait()

        def scatter(i, buf):
            pltpu.sync_copy(buf, agg_sh.at[dst_v.at[i]], add=True)

        issue(0, rows_a, sem_a)

        def body(j, carry):
            i0 = 2 * j
            issue(i0 + 1, rows_b, sem_b)
            wait(i0, rows_a, sem_a)
            scatter(i0, rows_a)
            issue(i0 + 2, rows_a, sem_a)
            wait(i0 + 1, rows_b, sem_b)
            scatter(i0 + 1, rows_b)
            return carry

        lax.fori_loop(0, (ITERS - 1) // 2, body, 0)
        wait(ITERS - 1, rows_a, sem_a)
        scatter(ITERS - 1, rows_a)

        plsc.subcore_barrier()

        for kk in range(2):
            cid = s + NS * kk

            @pl.when(cid < NRC)
            def _():
                off = cid * ROWC
                pltpu.sync_copy(agg_sh.at[pl.ds(off, ROWC)],
                                out_hbm.at[c, pl.ds(off, ROWC)])

    return k(h, src, dst3, zeros)


def _tc_mlp(agg, W1, b1, W2, b2):
    BLK = 2000
    grid = (N // BLK,)

    def body(a_ref, w1_ref, b1_ref, w2_ref, b2_ref, o_ref):
        y = a_ref[0] + a_ref[1]
        z = jnp.maximum(
            jnp.dot(y, w1_ref[...], preferred_element_type=jnp.float32,
                    precision=_HIGH) + b1_ref[...], 0.0)
        o_ref[...] = jnp.dot(z, w2_ref[...], preferred_element_type=jnp.float32,
                             precision=_HIGH) + b2_ref[...]

    return pl.pallas_call(
        body,
        grid=grid,
        in_specs=[
            pl.BlockSpec((NC, BLK, D), lambda i: (0, i, 0)),
            pl.BlockSpec((D, D), lambda i: (0, 0)),
            pl.BlockSpec((1, D), lambda i: (0, 0)),
            pl.BlockSpec((D, D), lambda i: (0, 0)),
            pl.BlockSpec((1, D), lambda i: (0, 0)),
        ],
        out_specs=pl.BlockSpec((BLK, D), lambda i: (i, 0)),
        out_shape=jax.ShapeDtypeStruct((N, D), jnp.float32),
    )(agg, W1, b1.reshape(1, D), W2, b2.reshape(1, D))


def _tc_mlp_pool(agg, W1, b1, W2, b2, batch3, mW1, mb1, mW2, mb2):
    BLK = 2000
    nblk = N // BLK

    def body(a_ref, w1_ref, b1_ref, w2_ref, b2_ref, bat_ref,
             pw1_ref, pb1_ref, pw2_ref, pb2_ref, o_ref, sums, counts):
        i = pl.program_id(0)

        @pl.when(i == 0)
        def _():
            sums[...] = jnp.zeros_like(sums)
            counts[...] = jnp.zeros_like(counts)

        y = a_ref[0] + a_ref[1]
        z = jnp.maximum(
            jnp.dot(y, w1_ref[...], preferred_element_type=jnp.float32,
                    precision=_HIGH) + b1_ref[...], 0.0)
        hblk = jnp.dot(z, w2_ref[...], preferred_element_type=jnp.float32,
                       precision=_HIGH) + b2_ref[...]

        ids = bat_ref[0, 0, :]
        onehot = (ids[None, :] == lax.broadcasted_iota(
            jnp.int32, (G, BLK), 0)).astype(jnp.float32)
        sums[...] += jnp.dot(onehot, hblk,
                             preferred_element_type=jnp.float32,
                             precision=_HIGH)
        counts[...] += jnp.sum(onehot, axis=1, keepdims=True)

        @pl.when(i == nblk - 1)
        def _():
            pooled = sums[...] / jnp.maximum(counts[...], 1.0)
            zz = jnp.maximum(
                jnp.dot(pooled, pw1_ref[...],
                        preferred_element_type=jnp.float32,
                        precision=_HIGH) + pb1_ref[...], 0.0)
            o_ref[...] = jnp.dot(zz, pw2_ref[...],
                                 preferred_element_type=jnp.float32,
                                 precision=_HIGH) + pb2_ref[...]

    return pl.pallas_call(
        body,
        grid=(nblk,),
        in_specs=[
            pl.BlockSpec((NC, BLK, D), lambda i: (0, i, 0)),
            pl.BlockSpec((D, D), lambda i: (0, 0)),
            pl.BlockSpec((1, D), lambda i: (0, 0)),
            pl.BlockSpec((D, D), lambda i: (0, 0)),
            pl.BlockSpec((1, D), lambda i: (0, 0)),
            pl.BlockSpec((1, 1, BLK), lambda i: (i, 0, 0)),
            pl.BlockSpec((D, D), lambda i: (0, 0)),
            pl.BlockSpec((1, D), lambda i: (0, 0)),
            pl.BlockSpec((D, G), lambda i: (0, 0)),
            pl.BlockSpec((1, G), lambda i: (0, 0)),
        ],
        out_specs=pl.BlockSpec((G, G), lambda i: (0, 0)),
        out_shape=jax.ShapeDtypeStruct((G, G), jnp.float32),
        scratch_shapes=[
            pltpu.VMEM((G, D), jnp.float32),
            pltpu.VMEM((G, 1), jnp.float32),
        ],
    )(agg, W1, b1.reshape(1, D), W2, b2.reshape(1, D), batch3,
      mW1, mb1.reshape(1, D), mW2, mb2.reshape(1, G))


def kernel(x, edge_index, batch, conv0_W1, conv0_b1, conv0_W2, conv0_b2,
           conv1_W1, conv1_b1, conv1_W2, conv1_b2,
           conv2_W1, conv2_b1, conv2_W2, conv2_b2,
           mlp_W1, mlp_b1, mlp_W2, mlp_b2):
    src = edge_index[0]
    dst3 = edge_index[1].reshape(NW, ITERS, CHUNK)
    zeros = jnp.zeros((N, D), dtype=jnp.float32)
    batch3 = batch.reshape(N // 2000, 1, 2000)

    h = x
    for (W1, b1, W2, b2) in (
        (conv0_W1, conv0_b1, conv0_W2, conv0_b2),
        (conv1_W1, conv1_b1, conv1_W2, conv1_b2),
    ):
        agg = _sc_aggregate(h, src, dst3, zeros)
        h = _tc_mlp(agg, W1, b1, W2, b2)

    agg = _sc_aggregate(h, src, dst3, zeros)
    return _tc_mlp_pool(agg, conv2_W1, conv2_b1, conv2_W2, conv2_b2,
                        batch3, mlp_W1, mlp_b1, mlp_W2, mlp_b2)


# speedup vs baseline: 13.4703x; 1.1994x over previous
import functools

import jax
import jax.numpy as jnp
from jax import lax
from jax.experimental import pallas as pl
from jax.experimental.pallas import tpu as pltpu
from jax.experimental.pallas import tpu_sc as plsc

N = 10000
E = 320000
D = 128
G = 128

NC = 2
NS = 16
NW = NC * NS
EP = E // NW
CHUNK = 80
ITERS = EP // CHUNK
ROWC = 400
NRC = N // ROWC

_HIGH = lax.Precision.HIGHEST


def _sc_aggregate(h, src, dst, zeros):
    mesh = plsc.VectorSubcoreMesh(core_axis_name="c", subcore_axis_name="s",
                                  num_cores=NC, num_subcores=NS)

    @functools.partial(
        pl.kernel,
        out_type=jax.ShapeDtypeStruct((NC, N, D), jnp.float32),
        mesh=mesh,
        scratch_types=[
            pltpu.VMEM((EP,), jnp.int32),
            pltpu.VMEM((3, CHUNK), jnp.int32),
            pltpu.VMEM((CHUNK, D), jnp.float32),
            pltpu.VMEM((CHUNK, D), jnp.float32),
            pltpu.VMEM((CHUNK, D), jnp.float32),
            pltpu.VMEM_SHARED((N, D), jnp.float32),
            pltpu.SemaphoreType.DMA,
            pltpu.SemaphoreType.DMA,
            pltpu.SemaphoreType.DMA,
            pltpu.SemaphoreType.DMA,
            pltpu.SemaphoreType.DMA,
            pltpu.SemaphoreType.DMA,
        ],
    )
    def k(h_hbm, src_hbm, dst_hbm, z_hbm, out_hbm, src_v, dst_v, rows_a,
          rows_b, rows_c, agg_sh, sem_a, sem_b, sem_c, sem_d0, sem_d1,
          sem_d2):
        c = lax.axis_index("c")
        s = lax.axis_index("s")
        wid = s * NC + c
        base = wid * EP

        rows = (rows_a, rows_b, rows_c)
        gsems = (sem_a, sem_b, sem_c)
        dsems = (sem_d0, sem_d1, sem_d2)

        def fire_dst(i, kk):
            pltpu.async_copy(dst_hbm.at[pl.ds(base + i * CHUNK, CHUNK)],
                             dst_v.at[kk], dsems[kk])

        def wait_dst(i, kk):
            pltpu.make_async_copy(dst_hbm.at[pl.ds(base + i * CHUNK, CHUNK)],
                                  dst_v.at[kk], dsems[kk]).wait()

        for kk in range(3):
            fire_dst(kk, kk)

        pltpu.sync_copy(src_hbm.at[pl.ds(base, EP)], src_v)

        for kk in range(2):
            cid = s + NS * kk

            @pl.when(cid < NRC)
            def _():
                off = cid * ROWC

                @pl.when(c == 0)
                def _():
                    pltpu.sync_copy(h_hbm.at[pl.ds(off, ROWC)],
                                    agg_sh.at[pl.ds(off, ROWC)])

                @pl.when(c != 0)
                def _():
                    pltpu.sync_copy(z_hbm.at[pl.ds(off, ROWC)],
                                    agg_sh.at[pl.ds(off, ROWC)])

        def issue(i, kk):
            pltpu.async_copy(h_hbm.at[src_v.at[pl.ds(i * CHUNK, CHUNK)]],
                             rows[kk], gsems[kk])

        def wait(i, kk):
            pltpu.make_async_copy(h_hbm.at[src_v.at[pl.ds(i * CHUNK, CHUNK)]],
                                  rows[kk], gsems[kk]).wait()

        def scatter(kk):
            pltpu.sync_copy(rows[kk], agg_sh.at[dst_v.at[kk]], add=True)

        for kk in range(3):
            issue(kk, kk)

        plsc.subcore_barrier()

        def body(j, carry):
            i0 = 3 * j
            for kk in range(3):
                i = i0 + kk
                wait_dst(i, kk)
                wait(i, kk)
                scatter(kk)

                @pl.when(i + 3 < ITERS)
                def _():
                    issue(i + 3, kk)
                    fire_dst(i + 3, kk)
            return carry

        lax.fori_loop(0, ITERS // 3, body, 0)
        for kk in range(2):
            i = (ITERS // 3) * 3 + kk
            wait_dst(i, kk)
            wait(i, kk)
            scatter(kk)

        plsc.subcore_barrier()

        for kk in range(2):
            cid = s + NS * kk

            @pl.when(cid < NRC)
            def _():
                off = cid * ROWC
                pltpu.sync_copy(agg_sh.at[pl.ds(off, ROWC)],
                                out_hbm.at[c, pl.ds(off, ROWC)])

    return k(h, src, dst, zeros)


def _tc_mlp(agg, W1, b1, W2, b2):
    BLK = 2000
    grid = (N // BLK,)

    def body(a_ref, w1_ref, b1_ref, w2_ref, b2_ref, o_ref):
        y = a_ref[0] + a_ref[1]
        z = jnp.maximum(
            jnp.dot(y, w1_ref[...], preferred_element_type=jnp.float32,
                    precision=_HIGH) + b1_ref[...], 0.0)
        o_ref[...] = jnp.dot(z, w2_ref[...], preferred_element_type=jnp.float32,
                             precision=_HIGH) + b2_ref[...]

    return pl.pallas_call(
        body,
        grid=grid,
        in_specs=[
            pl.BlockSpec((NC, BLK, D), lambda i: (0, i, 0)),
            pl.BlockSpec((D, D), lambda i: (0, 0)),
            pl.BlockSpec((1, D), lambda i: (0, 0)),
            pl.BlockSpec((D, D), lambda i: (0, 0)),
            pl.BlockSpec((1, D), lambda i: (0, 0)),
        ],
        out_specs=pl.BlockSpec((BLK, D), lambda i: (i, 0)),
        out_shape=jax.ShapeDtypeStruct((N, D), jnp.float32),
    )(agg, W1, b1.reshape(1, D), W2, b2.reshape(1, D))


def _tc_mlp_pool(agg, W1, b1, W2, b2, batch3, mW1, mb1, mW2, mb2):
    BLK = 2000
    nblk = N // BLK

    def body(a_ref, w1_ref, b1_ref, w2_ref, b2_ref, bat_ref,
             pw1_ref, pb1_ref, pw2_ref, pb2_ref, o_ref, sums, counts):
        i = pl.program_id(0)

        @pl.when(i == 0)
        def _():
            sums[...] = jnp.zeros_like(sums)
            counts[...] = jnp.zeros_like(counts)

        y = a_ref[0] + a_ref[1]
        z = jnp.maximum(
            jnp.dot(y, w1_ref[...], preferred_element_type=jnp.float32,
                    precision=_HIGH) + b1_ref[...], 0.0)
        hblk = jnp.dot(z, w2_ref[...], preferred_element_type=jnp.float32,
                       precision=_HIGH) + b2_ref[...]

        ids = bat_ref[0, 0, :]
        onehot = (ids[None, :] == lax.broadcasted_iota(
            jnp.int32, (G, BLK), 0)).astype(jnp.float32)
        sums[...] += jnp.dot(onehot, hblk,
                             preferred_element_type=jnp.float32,
                             precision=_HIGH)
        counts[...] += jnp.sum(onehot, axis=1, keepdims=True)

        @pl.when(i == nblk - 1)
        def _():
            pooled = sums[...] / jnp.maximum(counts[...], 1.0)
            zz = jnp.maximum(
                jnp.dot(pooled, pw1_ref[...],
                        preferred_element_type=jnp.float32,
                        precision=_HIGH) + pb1_ref[...], 0.0)
            o_ref[...] = jnp.dot(zz, pw2_ref[...],
                                 preferred_element_type=jnp.float32,
                                 precision=_HIGH) + pb2_ref[...]

    return pl.pallas_call(
        body,
        grid=(nblk,),
        in_specs=[
            pl.BlockSpec((NC, BLK, D), lambda i: (0, i, 0)),
            pl.BlockSpec((D, D), lambda i: (0, 0)),
            pl.BlockSpec((1, D), lambda i: (0, 0)),
            pl.BlockSpec((D, D), lambda i: (0, 0)),
            pl.BlockSpec((1, D), lambda i: (0, 0)),
            pl.BlockSpec((1, 1, BLK), lambda i: (i, 0, 0)),
            pl.BlockSpec((D, D), lambda i: (0, 0)),
            pl.BlockSpec((1, D), lambda i: (0, 0)),
            pl.BlockSpec((D, G), lambda i: (0, 0)),
            pl.BlockSpec((1, G), lambda i: (0, 0)),
        ],
        out_specs=pl.BlockSpec((G, G), lambda i: (0, 0)),
        out_shape=jax.ShapeDtypeStruct((G, G), jnp.float32),
        scratch_shapes=[
            pltpu.VMEM((G, D), jnp.float32),
            pltpu.VMEM((G, 1), jnp.float32),
        ],
    )(agg, W1, b1.reshape(1, D), W2, b2.reshape(1, D), batch3,
      mW1, mb1.reshape(1, D), mW2, mb2.reshape(1, G))


def kernel(x, edge_index, batch, conv0_W1, conv0_b1, conv0_W2, conv0_b2,
           conv1_W1, conv1_b1, conv1_W2, conv1_b2,
           conv2_W1, conv2_b1, conv2_W2, conv2_b2,
           mlp_W1, mlp_b1, mlp_W2, mlp_b2):
    src = edge_index[0]
    dst = edge_index[1]
    zeros = jnp.zeros((N, D), dtype=jnp.float32)
    batch3 = batch.reshape(N // 2000, 1, 2000)

    h = x
    for (W1, b1, W2, b2) in (
        (conv0_W1, conv0_b1, conv0_W2, conv0_b2),
        (conv1_W1, conv1_b1, conv1_W2, conv1_b2),
    ):
        agg = _sc_aggregate(h, src, dst, zeros)
        h = _tc_mlp(agg, W1, b1, W2, b2)

    agg = _sc_aggregate(h, src, dst, zeros)
    return _tc_mlp_pool(agg, conv2_W1, conv2_b1, conv2_W2, conv2_b2,
                        batch3, mlp_W1, mlp_b1, mlp_W2, mlp_b2)


# speedup vs baseline: 13.5815x; 1.0083x over previous
import functools

import jax
import jax.numpy as jnp
from jax import lax
from jax.experimental import pallas as pl
from jax.experimental.pallas import tpu as pltpu
from jax.experimental.pallas import tpu_sc as plsc

N = 10000
E = 320000
D = 128
G = 128

NC = 2
NS = 16
NW = NC * NS
EP = E // NW
CHUNK = 80
ITERS = EP // CHUNK
ROWC = 400
NRC = N // ROWC

_HIGH = lax.Precision.HIGHEST


def _sc_aggregate(h, src, dst, zeros):
    mesh = plsc.VectorSubcoreMesh(core_axis_name="c", subcore_axis_name="s",
                                  num_cores=NC, num_subcores=NS)

    @functools.partial(
        pl.kernel,
        out_type=jax.ShapeDtypeStruct((NC, N, D), jnp.float32),
        mesh=mesh,
        scratch_types=[
            pltpu.VMEM((EP,), jnp.int32),
            pltpu.VMEM((3, CHUNK), jnp.int32),
            pltpu.VMEM((CHUNK, D), jnp.float32),
            pltpu.VMEM((CHUNK, D), jnp.float32),
            pltpu.VMEM((CHUNK, D), jnp.float32),
            pltpu.VMEM_SHARED((N, D), jnp.float32),
            pltpu.SemaphoreType.DMA,
            pltpu.SemaphoreType.DMA,
            pltpu.SemaphoreType.DMA,
            pltpu.SemaphoreType.DMA,
            pltpu.SemaphoreType.DMA,
            pltpu.SemaphoreType.DMA,
        ],
    )
    def k(h_hbm, src_hbm, dst_hbm, z_hbm, out_hbm, src_v, dst_v, rows_a,
          rows_b, rows_c, agg_sh, sem_a, sem_b, sem_c, sem_d0, sem_d1,
          sem_d2):
        c = lax.axis_index("c")
        s = lax.axis_index("s")
        wid = s * NC + c
        base = wid * EP

        rows = (rows_a, rows_b, rows_c)
        gsems = (sem_a, sem_b, sem_c)
        dsems = (sem_d0, sem_d1, sem_d2)

        def fire_dst(i, kk):
            pltpu.async_copy(dst_hbm.at[pl.ds(base + i * CHUNK, CHUNK)],
                             dst_v.at[kk], dsems[kk])

        def wait_dst(i, kk):
            pltpu.make_async_copy(dst_hbm.at[pl.ds(base + i * CHUNK, CHUNK)],
                                  dst_v.at[kk], dsems[kk]).wait()

        for kk in range(3):
            fire_dst(kk, kk)

        pltpu.sync_copy(src_hbm.at[pl.ds(base, EP)], src_v)

        for kk in range(3):
            pltpu.async_copy(h_hbm.at[src_v.at[pl.ds(kk * CHUNK, CHUNK)]],
                             rows[kk], gsems[kk])

        for kk in range(2):
            cid = s + NS * kk

            @pl.when(cid < NRC)
            def _():
                off = cid * ROWC

                @pl.when(c == 0)
                def _():
                    pltpu.sync_copy(h_hbm.at[pl.ds(off, ROWC)],
                                    agg_sh.at[pl.ds(off, ROWC)])

                @pl.when(c != 0)
                def _():
                    pltpu.sync_copy(z_hbm.at[pl.ds(off, ROWC)],
                                    agg_sh.at[pl.ds(off, ROWC)])

        def issue(i, kk):
            pltpu.async_copy(h_hbm.at[src_v.at[pl.ds(i * CHUNK, CHUNK)]],
                             rows[kk], gsems[kk])

        def wait(i, kk):
            pltpu.make_async_copy(h_hbm.at[src_v.at[pl.ds(i * CHUNK, CHUNK)]],
                                  rows[kk], gsems[kk]).wait()

        def scatter(kk):
            pltpu.sync_copy(rows[kk], agg_sh.at[dst_v.at[kk]], add=True)

        plsc.subcore_barrier()

        def body(j, carry):
            i0 = 3 * j
            for kk in range(3):
                i = i0 + kk
                wait_dst(i, kk)
                wait(i, kk)
                scatter(kk)

                @pl.when(i + 3 < ITERS)
                def _():
                    issue(i + 3, kk)
                    fire_dst(i + 3, kk)
            return carry

        lax.fori_loop(0, ITERS // 3, body, 0)
        for kk in range(2):
            i = (ITERS // 3) * 3 + kk
            wait_dst(i, kk)
            wait(i, kk)
            scatter(kk)

        plsc.subcore_barrier()

        for kk in range(2):
            cid = s + NS * kk

            @pl.when(cid < NRC)
            def _():
                off = cid * ROWC
                pltpu.sync_copy(agg_sh.at[pl.ds(off, ROWC)],
                                out_hbm.at[c, pl.ds(off, ROWC)])

    return k(h, src, dst, zeros)


def _tc_mlp(agg, W1, b1, W2, b2):
    BLK = 2000
    grid = (N // BLK,)

    def body(a_ref, w1_ref, b1_ref, w2_ref, b2_ref, o_ref):
        y = a_ref[0] + a_ref[1]
        z = jnp.maximum(
            jnp.dot(y, w1_ref[...], preferred_element_type=jnp.float32,
                    precision=_HIGH) + b1_ref[...], 0.0)
        o_ref[...] = jnp.dot(z, w2_ref[...], preferred_element_type=jnp.float32,
                             precision=_HIGH) + b2_ref[...]

    return pl.pallas_call(
        body,
        grid=grid,
        in_specs=[
            pl.BlockSpec((NC, BLK, D), lambda i: (0, i, 0)),
            pl.BlockSpec((D, D), lambda i: (0, 0)),
            pl.BlockSpec((1, D), lambda i: (0, 0)),
            pl.BlockSpec((D, D), lambda i: (0, 0)),
            pl.BlockSpec((1, D), lambda i: (0, 0)),
        ],
        out_specs=pl.BlockSpec((BLK, D), lambda i: (i, 0)),
        out_shape=jax.ShapeDtypeStruct((N, D), jnp.float32),
    )(agg, W1, b1.reshape(1, D), W2, b2.reshape(1, D))


def _tc_mlp_pool(agg, W1, b1, W2, b2, batch3, mW1, mb1, mW2, mb2):
    BLK = 2000
    nblk = N // BLK

    def body(a_ref, w1_ref, b1_ref, w2_ref, b2_ref, bat_ref,
             pw1_ref, pb1_ref, pw2_ref, pb2_ref, o_ref, sums, counts):
        i = pl.program_id(0)

        @pl.when(i == 0)
        def _():
            sums[...] = jnp.zeros_like(sums)
            counts[...] = jnp.zeros_like(counts)

        y = a_ref[0] + a_ref[1]
        z = jnp.maximum(
            jnp.dot(y, w1_ref[...], preferred_element_type=jnp.float32,
                    precision=_HIGH) + b1_ref[...], 0.0)
        hblk = jnp.dot(z, w2_ref[...], preferred_element_type=jnp.float32,
                       precision=_HIGH) + b2_ref[...]

        ids = bat_ref[0, 0, :]
        onehot = (ids[None, :] == lax.broadcasted_iota(
            jnp.int32, (G, BLK), 0)).astype(jnp.float32)
        sums[...] += jnp.dot(onehot, hblk,
                             preferred_element_type=jnp.float32,
                             precision=_HIGH)
        counts[...] += jnp.sum(onehot, axis=1, keepdims=True)

        @pl.when(i == nblk - 1)
        def _():
            pooled = sums[...] / jnp.maximum(counts[...], 1.0)
            zz = jnp.maximum(
                jnp.dot(pooled, pw1_ref[...],
                        preferred_element_type=jnp.float32,
                        precision=_HIGH) + pb1_ref[...], 0.0)
            o_ref[...] = jnp.dot(zz, pw2_ref[...],
                                 preferred_element_type=jnp.float32,
                                 precision=_HIGH) + pb2_ref[...]

    return pl.pallas_call(
        body,
        grid=(nblk,),
        in_specs=[
            pl.BlockSpec((NC, BLK, D), lambda i: (0, i, 0)),
            pl.BlockSpec((D, D), lambda i: (0, 0)),
            pl.BlockSpec((1, D), lambda i: (0, 0)),
            pl.BlockSpec((D, D), lambda i: (0, 0)),
            pl.BlockSpec((1, D), lambda i: (0, 0)),
            pl.BlockSpec((1, 1, BLK), lambda i: (i, 0, 0)),
            pl.BlockSpec((D, D), lambda i: (0, 0)),
            pl.BlockSpec((1, D), lambda i: (0, 0)),
            pl.BlockSpec((D, G), lambda i: (0, 0)),
            pl.BlockSpec((1, G), lambda i: (0, 0)),
        ],
        out_specs=pl.BlockSpec((G, G), lambda i: (0, 0)),
        out_shape=jax.ShapeDtypeStruct((G, G), jnp.float32),
        scratch_shapes=[
            pltpu.VMEM((G, D), jnp.float32),
            pltpu.VMEM((G, 1), jnp.float32),
        ],
    )(agg, W1, b1.reshape(1, D), W2, b2.reshape(1, D), batch3,
      mW1, mb1.reshape(1, D), mW2, mb2.reshape(1, G))


def kernel(x, edge_index, batch, conv0_W1, conv0_b1, conv0_W2, conv0_b2,
           conv1_W1, conv1_b1, conv1_W2, conv1_b2,
           conv2_W1, conv2_b1, conv2_W2, conv2_b2,
           mlp_W1, mlp_b1, mlp_W2, mlp_b2):
    src = edge_index[0]
    dst = edge_index[1]
    zeros = jnp.zeros((N, D), dtype=jnp.float32)
    batch3 = batch.reshape(N // 2000, 1, 2000)

    h = x
    for (W1, b1, W2, b2) in (
        (conv0_W1, conv0_b1, conv0_W2, conv0_b2),
        (conv1_W1, conv1_b1, conv1_W2, conv1_b2),
    ):
        agg = _sc_aggregate(h, src, dst, zeros)
        h = _tc_mlp(agg, W1, b1, W2, b2)

    agg = _sc_aggregate(h, src, dst, zeros)
    return _tc_mlp_pool(agg, conv2_W1, conv2_b1, conv2_W2, conv2_b2,
                        batch3, mlp_W1, mlp_b1, mlp_W2, mlp_b2)


# speedup vs baseline: 15.0451x; 1.1078x over previous
import functools

import jax
import jax.numpy as jnp
from jax import lax
from jax.experimental import pallas as pl
from jax.experimental.pallas import tpu as pltpu
from jax.experimental.pallas import tpu_sc as plsc

N = 10000
E = 320000
D = 128
G = 128

NC = 2
NS = 16
NW = NC * NS
EP = E // NW
CHUNK = 80
ITERS = EP // CHUNK
ROWC = 400
NRC = N // ROWC

_HIGH = lax.Precision.DEFAULT


def _sc_aggregate(h, src, dst, zeros):
    mesh = plsc.VectorSubcoreMesh(core_axis_name="c", subcore_axis_name="s",
                                  num_cores=NC, num_subcores=NS)

    @functools.partial(
        pl.kernel,
        out_type=jax.ShapeDtypeStruct((NC, N, D), jnp.float32),
        mesh=mesh,
        scratch_types=[
            pltpu.VMEM((EP,), jnp.int32),
            pltpu.VMEM((3, CHUNK), jnp.int32),
            pltpu.VMEM((CHUNK, D), jnp.float32),
            pltpu.VMEM((CHUNK, D), jnp.float32),
            pltpu.VMEM((CHUNK, D), jnp.float32),
            pltpu.VMEM_SHARED((N, D), jnp.float32),
            pltpu.SemaphoreType.DMA,
            pltpu.SemaphoreType.DMA,
            pltpu.SemaphoreType.DMA,
            pltpu.SemaphoreType.DMA,
            pltpu.SemaphoreType.DMA,
            pltpu.SemaphoreType.DMA,
        ],
    )
    def k(h_hbm, src_hbm, dst_hbm, z_hbm, out_hbm, src_v, dst_v, rows_a,
          rows_b, rows_c, agg_sh, sem_a, sem_b, sem_c, sem_d0, sem_d1,
          sem_d2):
        c = lax.axis_index("c")
        s = lax.axis_index("s")
        wid = s * NC + c
        base = wid * EP

        rows = (rows_a, rows_b, rows_c)
        gsems = (sem_a, sem_b, sem_c)
        dsems = (sem_d0, sem_d1, sem_d2)

        def fire_dst(i, kk):
            pltpu.async_copy(dst_hbm.at[pl.ds(base + i * CHUNK, CHUNK)],
                             dst_v.at[kk], dsems[kk])

        def wait_dst(i, kk):
            pltpu.make_async_copy(dst_hbm.at[pl.ds(base + i * CHUNK, CHUNK)],
                                  dst_v.at[kk], dsems[kk]).wait()

        for kk in range(3):
            fire_dst(kk, kk)

        pltpu.sync_copy(src_hbm.at[pl.ds(base, EP)], src_v)

        for kk in range(3):
            pltpu.async_copy(h_hbm.at[src_v.at[pl.ds(kk * CHUNK, CHUNK)]],
                             rows[kk], gsems[kk])

        for kk in range(2):
            cid = s + NS * kk

            @pl.when(cid < NRC)
            def _():
                off = cid * ROWC

                @pl.when(c == 0)
                def _():
                    pltpu.sync_copy(h_hbm.at[pl.ds(off, ROWC)],
                                    agg_sh.at[pl.ds(off, ROWC)])

                @pl.when(c != 0)
                def _():
                    pltpu.sync_copy(z_hbm.at[pl.ds(off, ROWC)],
                                    agg_sh.at[pl.ds(off, ROWC)])

        def issue(i, kk):
            pltpu.async_copy(h_hbm.at[src_v.at[pl.ds(i * CHUNK, CHUNK)]],
                             rows[kk], gsems[kk])

        def wait(i, kk):
            pltpu.make_async_copy(h_hbm.at[src_v.at[pl.ds(i * CHUNK, CHUNK)]],
                                  rows[kk], gsems[kk]).wait()

        def scatter(kk):
            pltpu.sync_copy(rows[kk], agg_sh.at[dst_v.at[kk]], add=True)

        plsc.subcore_barrier()

        def body(j, carry):
            i0 = 3 * j
            for kk in range(3):
                i = i0 + kk
                wait_dst(i, kk)
                wait(i, kk)
                scatter(kk)

                @pl.when(i + 3 < ITERS)
                def _():
                    issue(i + 3, kk)
                    fire_dst(i + 3, kk)
            return carry

        lax.fori_loop(0, ITERS // 3, body, 0)
        for kk in range(2):
            i = (ITERS // 3) * 3 + kk
            wait_dst(i, kk)
            wait(i, kk)
            scatter(kk)

        plsc.subcore_barrier()

        for kk in range(2):
            cid = s + NS * kk

            @pl.when(cid < NRC)
            def _():
                off = cid * ROWC
                pltpu.sync_copy(agg_sh.at[pl.ds(off, ROWC)],
                                out_hbm.at[c, pl.ds(off, ROWC)])

    return k(h, src, dst, zeros)


def _tc_mlp(agg, W1, b1, W2, b2):
    BLK = 2000
    grid = (N // BLK,)

    def body(a_ref, w1_ref, b1_ref, w2_ref, b2_ref, o_ref):
        y = a_ref[0] + a_ref[1]
        z = jnp.maximum(
            jnp.dot(y, w1_ref[...], preferred_element_type=jnp.float32,
                    precision=_HIGH) + b1_ref[...], 0.0)
        o_ref[...] = jnp.dot(z, w2_ref[...], preferred_element_type=jnp.float32,
                             precision=_HIGH) + b2_ref[...]

    return pl.pallas_call(
        body,
        grid=grid,
        in_specs=[
            pl.BlockSpec((NC, BLK, D), lambda i: (0, i, 0)),
            pl.BlockSpec((D, D), lambda i: (0, 0)),
            pl.BlockSpec((1, D), lambda i: (0, 0)),
            pl.BlockSpec((D, D), lambda i: (0, 0)),
            pl.BlockSpec((1, D), lambda i: (0, 0)),
        ],
        out_specs=pl.BlockSpec((BLK, D), lambda i: (i, 0)),
        out_shape=jax.ShapeDtypeStruct((N, D), jnp.float32),
    )(agg, W1, b1.reshape(1, D), W2, b2.reshape(1, D))


def _tc_mlp_pool(agg, W1, b1, W2, b2, batch3, mW1, mb1, mW2, mb2):
    BLK = 2000
    nblk = N // BLK

    def body(a_ref, w1_ref, b1_ref, w2_ref, b2_ref, bat_ref,
             pw1_ref, pb1_ref, pw2_ref, pb2_ref, o_ref, sums, counts):
        i = pl.program_id(0)

        @pl.when(i == 0)
        def _():
            sums[...] = jnp.zeros_like(sums)
            counts[...] = jnp.zeros_like(counts)

        y = a_ref[0] + a_ref[1]
        z = jnp.maximum(
            jnp.dot(y, w1_ref[...], preferred_element_type=jnp.float32,
                    precision=_HIGH) + b1_ref[...], 0.0)
        hblk = jnp.dot(z, w2_ref[...], preferred_element_type=jnp.float32,
                       precision=_HIGH) + b2_ref[...]

        ids = bat_ref[0, 0, :]
        onehot = (ids[None, :] == lax.broadcasted_iota(
            jnp.int32, (G, BLK), 0)).astype(jnp.float32)
        sums[...] += jnp.dot(onehot, hblk,
                             preferred_element_type=jnp.float32,
                             precision=_HIGH)
        counts[...] += jnp.sum(onehot, axis=1, keepdims=True)

        @pl.when(i == nblk - 1)
        def _():
            pooled = sums[...] / jnp.maximum(counts[...], 1.0)
            zz = jnp.maximum(
                jnp.dot(pooled, pw1_ref[...],
                        preferred_element_type=jnp.float32,
                        precision=_HIGH) + pb1_ref[...], 0.0)
            o_ref[...] = jnp.dot(zz, pw2_ref[...],
                                 preferred_element_type=jnp.float32,
                                 precision=_HIGH) + pb2_ref[...]

    return pl.pallas_call(
        body,
        grid=(nblk,),
        in_specs=[
            pl.BlockSpec((NC, BLK, D), lambda i: (0, i, 0)),
            pl.BlockSpec((D, D), lambda i: (0, 0)),
            pl.BlockSpec((1, D), lambda i: (0, 0)),
            pl.BlockSpec((D, D), lambda i: (0, 0)),
            pl.BlockSpec((1, D), lambda i: (0, 0)),
            pl.BlockSpec((1, 1, BLK), lambda i: (i, 0, 0)),
            pl.BlockSpec((D, D), lambda i: (0, 0)),
            pl.BlockSpec((1, D), lambda i: (0, 0)),
            pl.BlockSpec((D, G), lambda i: (0, 0)),
            pl.BlockSpec((1, G), lambda i: (0, 0)),
        ],
        out_specs=pl.BlockSpec((G, G), lambda i: (0, 0)),
        out_shape=jax.ShapeDtypeStruct((G, G), jnp.float32),
        scratch_shapes=[
            pltpu.VMEM((G, D), jnp.float32),
            pltpu.VMEM((G, 1), jnp.float32),
        ],
    )(agg, W1, b1.reshape(1, D), W2, b2.reshape(1, D), batch3,
      mW1, mb1.reshape(1, D), mW2, mb2.reshape(1, G))


def kernel(x, edge_index, batch, conv0_W1, conv0_b1, conv0_W2, conv0_b2,
           conv1_W1, conv1_b1, conv1_W2, conv1_b2,
           conv2_W1, conv2_b1, conv2_W2, conv2_b2,
           mlp_W1, mlp_b1, mlp_W2, mlp_b2):
    src = edge_index[0]
    dst = edge_index[1]
    zeros = jnp.zeros((N, D), dtype=jnp.float32)
    batch3 = batch.reshape(N // 2000, 1, 2000)

    h = x
    for (W1, b1, W2, b2) in (
        (conv0_W1, conv0_b1, conv0_W2, conv0_b2),
        (conv1_W1, conv1_b1, conv1_W2, conv1_b2),
    ):
        agg = _sc_aggregate(h, src, dst, zeros)
        h = _tc_mlp(agg, W1, b1, W2, b2)

    agg = _sc_aggregate(h, src, dst, zeros)
    return _tc_mlp_pool(agg, conv2_W1, conv2_b1, conv2_W2, conv2_b2,
                        batch3, mlp_W1, mlp_b1, mlp_W2, mlp_b2)
